# Initial kernel scaffold; baseline (speedup 1.0000x reference)
import jax
import jax.numpy as jnp
from jax.experimental import pallas as pl


def kernel(price_input, e, concept, volumn, Wih, Whh, bih, bhh, W1, b1, W2, b2, Wl1, bl1):
    raise NotImplementedError("write your pallas kernel here")


# speedup vs baseline: 6.3710x; 6.3710x over previous
import functools

import jax
import jax.numpy as jnp
from jax import lax
from jax.experimental import pallas as pl
from jax.experimental.pallas import tpu as pltpu
from jax.experimental.pallas import tpu_sc as plsc

N = 10000
E = 160000
SEQ = 64
FP = 8
H = 128
N_OUT = 5
HO = 128

NC, NS, L = 2, 16, 16
NW = NC * NS
CH = 128
NCHUNK = E // CH
BASE_TRIPS = NCHUNK // NW
EXTRA = NCHUNK % NW
SR = 624
TAIL = N - NS * SR
ZCH = ((0, 128), (128, 128), (256, 128), (384, 128), (512, 112))

BN = 1000
BNG = 1024
NPAD = 10240


def _sigmoid(x):
    return 1.0 / (1.0 + jnp.exp(-x))


def _leaky(x):
    return jnp.where(x >= 0, x, 0.01 * x)


def _gru_body(x_ref, wih_ref, whh_ref, bih_ref, bhh_ref, w1_ref, out_ref, h_ref):
    h_ref[...] = jnp.zeros_like(h_ref)

    def step(t, _):
        xt_c = x_ref[pl.ds(t * FP, FP), :]
        gi = lax.dot_general(xt_c, wih_ref[...], (((0,), (0,)), ((), ())),
                             preferred_element_type=jnp.float32) + bih_ref[...]
        h = h_ref[...]
        gh = jnp.dot(h, whh_ref[...], preferred_element_type=jnp.float32) + bhh_ref[...]
        r = _sigmoid(gi[:, :H] + gh[:, :H])
        z = _sigmoid(gi[:, H:2 * H] + gh[:, H:2 * H])
        n = jnp.tanh(gi[:, 2 * H:] + r * gh[:, 2 * H:])
        h_ref[...] = (1.0 - z) * n + z * h
        return 0

    lax.fori_loop(0, SEQ, step, 0)
    out_ref[...] = jnp.dot(h_ref[...], w1_ref[...], preferred_element_type=jnp.float32)


def _gru_xw1(xTF, wihT, whhT, bih2, bhh2, w1T):
    return pl.pallas_call(
        _gru_body,
        grid=(NPAD // BNG,),
        in_specs=[
            pl.BlockSpec((SEQ * FP, BNG), lambda i: (0, i)),
            pl.BlockSpec((FP, 3 * H), lambda i: (0, 0)),
            pl.BlockSpec((H, 3 * H), lambda i: (0, 0)),
            pl.BlockSpec((1, 3 * H), lambda i: (0, 0)),
            pl.BlockSpec((1, 3 * H), lambda i: (0, 0)),
            pl.BlockSpec((H, H), lambda i: (0, 0)),
        ],
        out_specs=pl.BlockSpec((BNG, H), lambda i: (i, 0)),
        out_shape=jax.ShapeDtypeStruct((N, H), jnp.float32),
        scratch_shapes=[pltpu.VMEM((BNG, H), jnp.float32)],
    )(xTF, wihT, whhT, bih2, bhh2, w1T)


CW = 16


def _counts(node_idx, edge_idx):
    mesh = plsc.VectorSubcoreMesh(
        core_axis_name="c", subcore_axis_name="s", num_cores=NC, num_subcores=NS)

    @functools.partial(
        pl.kernel,
        out_type=(jax.ShapeDtypeStruct((NC, N, CW), jnp.float32),
                  jax.ShapeDtypeStruct((NC, N, CW), jnp.float32)),
        mesh=mesh,
        scratch_types=[
            pltpu.VMEM((CH,), jnp.int32),
            pltpu.VMEM((CH,), jnp.int32),
            pltpu.VMEM((CH, CW), jnp.float32),
            pltpu.VMEM((CH, CW), jnp.float32),
            pltpu.VMEM_SHARED((N, CW), jnp.float32),
            pltpu.VMEM_SHARED((N, CW), jnp.float32),
        ],
        compiler_params=pltpu.CompilerParams(use_tc_tiling_on_sc=False),
    )
    def k(src_hbm, dst_hbm, on_hbm, oe_hbm, sidx_v, didx_v, ones_v, zero_v,
          accn_sh, acce_sh):
        cid = lax.axis_index("c")
        sid = lax.axis_index("s")
        wid = sid * NC + cid
        ones16 = jnp.full((L,), 1.0, jnp.float32)
        zeros16 = jnp.zeros((L,), jnp.float32)

        def fill(i, _):
            ones_v[i] = ones16
            zero_v[i] = zeros16
            return 0

        lax.fori_loop(0, CH, fill, 0)
        base = sid * SR
        for o, sz in ZCH:
            pltpu.sync_copy(zero_v.at[pl.ds(0, sz)], accn_sh.at[pl.ds(base + o, sz)])
            pltpu.sync_copy(zero_v.at[pl.ds(0, sz)], acce_sh.at[pl.ds(base + o, sz)])

        @pl.when(sid == NS - 1)
        def _zero_tail():
            pltpu.sync_copy(zero_v.at[pl.ds(0, TAIL)], accn_sh.at[pl.ds(NS * SR, TAIL)])
            pltpu.sync_copy(zero_v.at[pl.ds(0, TAIL)], acce_sh.at[pl.ds(NS * SR, TAIL)])

        plsc.subcore_barrier()

        trips = BASE_TRIPS + jnp.where(wid < EXTRA, 1, 0)

        def chunk(j, _):
            off = (wid + NW * j) * CH
            pltpu.sync_copy(src_hbm.at[pl.ds(off, CH)], sidx_v)
            pltpu.sync_copy(dst_hbm.at[pl.ds(off, CH)], didx_v)
            pltpu.sync_copy(ones_v, accn_sh.at[sidx_v], add=True)
            pltpu.sync_copy(ones_v, acce_sh.at[didx_v], add=True)
            return 0

        lax.fori_loop(0, trips, chunk, 0)
        plsc.subcore_barrier()
        pltpu.sync_copy(accn_sh.at[pl.ds(base, SR)], on_hbm.at[cid, pl.ds(base, SR)])
        pltpu.sync_copy(acce_sh.at[pl.ds(base, SR)], oe_hbm.at[cid, pl.ds(base, SR)])

        @pl.when(sid == NS - 1)
        def _out_tail():
            pltpu.sync_copy(accn_sh.at[pl.ds(NS * SR, TAIL)],
                            on_hbm.at[cid, pl.ds(NS * SR, TAIL)])
            pltpu.sync_copy(acce_sh.at[pl.ds(NS * SR, TAIL)],
                            oe_hbm.at[cid, pl.ds(NS * SR, TAIL)])

    return k(node_idx, edge_idx)


def _spmm(table, src_idx, dst_idx):
    mesh = plsc.VectorSubcoreMesh(
        core_axis_name="c", subcore_axis_name="s", num_cores=NC, num_subcores=NS)

    @functools.partial(
        pl.kernel,
        out_type=jax.ShapeDtypeStruct((NC, N, H), jnp.float32),
        mesh=mesh,
        scratch_types=[
            pltpu.VMEM((CH,), jnp.int32),
            pltpu.VMEM((CH,), jnp.int32),
            pltpu.VMEM((CH, H), jnp.float32),
            pltpu.VMEM_SHARED((N, H), jnp.float32),
            pltpu.SemaphoreType.DMA,
        ],
    )
    def k(table_hbm, src_hbm, dst_hbm, out_hbm, sidx_v, didx_v, rows_v, acc_sh, sem):
        cid = lax.axis_index("c")
        sid = lax.axis_index("s")
        wid = sid * NC + cid
        zeros16 = jnp.zeros((L,), jnp.float32)

        def zloop(i, _):
            r = i // (H // L)
            c = (i % (H // L)) * L
            rows_v[r, pl.ds(c, L)] = zeros16
            return 0

        lax.fori_loop(0, CH * H // L, zloop, 0)
        base = sid * SR
        for o, sz in ZCH:
            pltpu.sync_copy(rows_v.at[pl.ds(0, sz)], acc_sh.at[pl.ds(base + o, sz)])

        @pl.when(sid == NS - 1)
        def _zero_tail():
            pltpu.sync_copy(rows_v.at[pl.ds(0, TAIL)], acc_sh.at[pl.ds(NS * SR, TAIL)])

        plsc.subcore_barrier()

        trips = BASE_TRIPS + jnp.where(wid < EXTRA, 1, 0)

        def chunk(j, _):
            off = (wid + NW * j) * CH
            pltpu.sync_copy(src_hbm.at[pl.ds(off, CH)], sidx_v)
            pltpu.sync_copy(dst_hbm.at[pl.ds(off, CH)], didx_v)
            pltpu.async_copy(table_hbm.at[sidx_v], rows_v, sem).wait()
            pltpu.sync_copy(rows_v, acc_sh.at[didx_v], add=True)
            return 0

        lax.fori_loop(0, trips, chunk, 0)
        plsc.subcore_barrier()
        pltpu.sync_copy(acc_sh.at[pl.ds(base, SR)], out_hbm.at[cid, pl.ds(base, SR)])

        @pl.when(sid == NS - 1)
        def _out_tail():
            pltpu.sync_copy(acc_sh.at[pl.ds(NS * SR, TAIL)],
                            out_hbm.at[cid, pl.ds(NS * SR, TAIL)])

    return k(table, src_idx, dst_idx)


def _degs_body(cn_ref, ce_ref, dn_ref, de_ref):
    dsum = cn_ref[0, :, 0:1] + cn_ref[1, :, 0:1]
    esum = ce_ref[0, :, 0:1] + ce_ref[1, :, 0:1]
    dn_ref[...] = jnp.where(dsum > 0, 1.0 / jnp.where(dsum > 0, dsum, 1.0), 0.0)
    de_ref[...] = jnp.where(esum > 0, 1.0 / jnp.where(esum > 0, esum, 1.0), 0.0)


def _degs(cn, ce):
    return pl.pallas_call(
        _degs_body,
        out_shape=(jax.ShapeDtypeStruct((N, 1), jnp.float32),
                   jax.ShapeDtypeStruct((N, 1), jnp.float32)),
    )(cn, ce)


def _scale_body(p_ref, s_ref, out_ref):
    out_ref[...] = s_ref[...] * (p_ref[0] + p_ref[1])


def _scale(p, s):
    return pl.pallas_call(
        _scale_body,
        grid=(N // BN,),
        in_specs=[
            pl.BlockSpec((NC, BN, H), lambda i: (0, i, 0)),
            pl.BlockSpec((BN, 1), lambda i: (i, 0)),
        ],
        out_specs=pl.BlockSpec((BN, H), lambda i: (i, 0)),
        out_shape=jax.ShapeDtypeStruct((N, H), jnp.float32),
    )(p, s)


def _mid_body(p_ref, s_ref, b_ref, w_ref, out_ref):
    v = s_ref[...] * (p_ref[0] + p_ref[1]) + b_ref[...]
    x1 = _leaky(v)
    out_ref[...] = jnp.dot(x1, w_ref[...], preferred_element_type=jnp.float32)


def _mid(p, s, b, wT):
    return pl.pallas_call(
        _mid_body,
        grid=(N // BN,),
        in_specs=[
            pl.BlockSpec((NC, BN, H), lambda i: (0, i, 0)),
            pl.BlockSpec((BN, 1), lambda i: (i, 0)),
            pl.BlockSpec((1, H), lambda i: (0, 0)),
            pl.BlockSpec((H, H), lambda i: (0, 0)),
        ],
        out_specs=pl.BlockSpec((BN, H), lambda i: (i, 0)),
        out_shape=jax.ShapeDtypeStruct((N, H), jnp.float32),
    )(p, s, b, wT)


def _final_body(p_ref, s_ref, b_ref, w_ref, bl_ref, out_ref):
    v = s_ref[...] * (p_ref[0] + p_ref[1]) + b_ref[...]
    x2 = _leaky(v)
    y = jnp.dot(x2, w_ref[...], preferred_element_type=jnp.float32) + bl_ref[...]
    out_ref[...] = _leaky(y)


def _final(p, s, b, wT, bl):
    return pl.pallas_call(
        _final_body,
        grid=(N // BN,),
        in_specs=[
            pl.BlockSpec((NC, BN, H), lambda i: (0, i, 0)),
            pl.BlockSpec((BN, 1), lambda i: (i, 0)),
            pl.BlockSpec((1, H), lambda i: (0, 0)),
            pl.BlockSpec((H, HO), lambda i: (0, 0)),
            pl.BlockSpec((1, HO), lambda i: (0, 0)),
        ],
        out_specs=pl.BlockSpec((BN, HO), lambda i: (i, 0)),
        out_shape=jax.ShapeDtypeStruct((N, HO), jnp.float32),
    )(p, s, b, wT, bl)


def kernel(price_input, e, concept, volumn, Wih, Whh, bih, bhh, W1, b1, W2, b2, Wl1, bl1):
    del concept, volumn
    xp = jnp.pad(price_input, ((0, 0), (0, 0), (0, FP - price_input.shape[-1])))
    xTF = jnp.transpose(xp, (1, 2, 0)).reshape(SEQ * FP, N)
    xTF = jnp.pad(xTF, ((0, 0), (0, NPAD - N)))
    wihT = jnp.pad(Wih.T, ((0, FP - Wih.shape[1]), (0, 0)))
    whhT = Whh.T
    bih2 = bih.reshape(1, -1)
    bhh2 = bhh.reshape(1, -1)

    node_idx = e[0]
    edge_idx = e[1]

    xw1 = _gru_xw1(xTF, wihT, whhT, bih2, bhh2, W1.T)

    cn, ce = _counts(node_idx, edge_idx)
    dinv_c, binv_c = _degs(cn, ce)

    p1 = _spmm(xw1, node_idx, edge_idx)
    ef1 = _scale(p1, binv_c)
    p2 = _spmm(ef1, edge_idx, node_idx)
    xw2 = _mid(p2, dinv_c, b1.reshape(1, -1), W2.T)

    p3 = _spmm(xw2, node_idx, edge_idx)
    ef2 = _scale(p3, binv_c)
    p4 = _spmm(ef2, edge_idx, node_idx)

    wl1T = jnp.pad(Wl1.T, ((0, 0), (0, HO - N_OUT)))
    bl1p = jnp.pad(bl1, (0, HO - N_OUT)).reshape(1, -1)
    y = _final(p4, dinv_c, b2.reshape(1, -1), wl1T, bl1p)
    return y[:, :N_OUT]
